# baseline (speedup 1.0000x reference)
import functools

import jax
import jax.numpy as jnp
from jax import lax
from jax.experimental import pallas as pl
from jax.experimental.pallas import tpu as pltpu
from jax.experimental.pallas import tpu_sc as plsc

N_TOK = 16 * 576
K = 8192
D = 256

TN = 512
TK = 1024
NB = N_TOK // TN
KB = K // TK

SC_NC = 2
SC_NS = 16
SC_NW = SC_NC * SC_NS
BPW = N_TOK // SC_NW


def _argmin_body(xsq_ref, x_ref, e_ref, esq_ref, out_ref, best_val, best_idx):
    j = pl.program_id(1)

    @pl.when(j == 0)
    def _init():
        best_val[...] = jnp.full((1, TN), -jnp.inf, jnp.float32)
        best_idx[...] = jnp.zeros((1, TN), jnp.int32)

    x = x_ref[...]
    e = e_ref[...]
    xe = lax.dot_general(x, e, (((1,), (1,)), ((), ())),
                         preferred_element_type=jnp.float32)
    dist = -((xsq_ref[...] - 2.0 * xe) + esq_ref[...])
    m = jnp.max(dist, axis=1)
    iota = lax.broadcasted_iota(jnp.int32, (TN, TK), 1)
    loc = jnp.min(jnp.where(dist == m[:, None], iota, TK), axis=1)
    cand = loc + j * TK
    prev_v = best_val[0, :]
    prev_i = best_idx[0, :]
    better = m > prev_v
    best_val[0, :] = jnp.where(better, m, prev_v)
    best_idx[0, :] = jnp.where(better, cand, prev_i)

    @pl.when(j == KB - 1)
    def _emit():
        out_ref[0, 0, :] = best_idx[0, :]


_argmin_call = pl.pallas_call(
    _argmin_body,
    grid=(NB, KB),
    in_specs=[
        pl.BlockSpec((TN, 1), lambda i, j: (i, 0)),
        pl.BlockSpec((TN, D), lambda i, j: (i, 0)),
        pl.BlockSpec((TK, D), lambda i, j: (j, 0)),
        pl.BlockSpec((1, TK), lambda i, j: (0, j)),
    ],
    out_specs=pl.BlockSpec((1, 1, TN), lambda i, j: (i, 0, 0)),
    out_shape=jax.ShapeDtypeStruct((NB, 1, TN), jnp.int32),
    scratch_shapes=[
        pltpu.VMEM((1, TN), jnp.float32),
        pltpu.VMEM((1, TN), jnp.int32),
    ],
)


@functools.lru_cache(maxsize=1)
def _sc_gather():
    mesh = plsc.VectorSubcoreMesh(
        core_axis_name="c", subcore_axis_name="s",
        num_cores=SC_NC, num_subcores=SC_NS)

    @functools.partial(
        pl.kernel,
        mesh=mesh,
        out_type=jax.ShapeDtypeStruct((N_TOK, D), jnp.float32),
        scratch_types=[
            pltpu.VMEM((BPW,), jnp.int32),
            pltpu.VMEM((BPW, D), jnp.float32),
            pltpu.SemaphoreType.DMA,
        ],
    )
    def gather(table_hbm, idx_hbm, out_hbm, idx_v, rows_v, sem):
        wid = lax.axis_index("s") * SC_NC + lax.axis_index("c")
        base = wid * BPW
        pltpu.sync_copy(idx_hbm.at[pl.ds(base, BPW)], idx_v)
        pltpu.async_copy(table_hbm.at[idx_v], rows_v, sem).wait()
        pltpu.sync_copy(rows_v, out_hbm.at[pl.ds(base, BPW)])

    return gather


def kernel(x, embed):
    flatten = x.reshape(N_TOK, D)
    table = embed[0]
    x_sq = jnp.sum(flatten ** 2, axis=-1, keepdims=True)
    e_sq = jnp.sum(embed ** 2, axis=-1)
    idx = _argmin_call(x_sq, flatten, table, e_sq).reshape(N_TOK)
    quantize = _sc_gather()(table, idx)
    return quantize.reshape(x.shape), idx.reshape(x.shape[:-1])
